# baseline (speedup 1.0000x reference)
import functools

import jax
import jax.numpy as jnp
from jax import lax
from jax.experimental import pallas as pl
from jax.experimental.pallas import tpu as pltpu
from jax.experimental.pallas import tpu_sc as plsc

_N_NODES = 10000
_N_EDGES = 320000
_D = 128
_NSEG = 10000
_NC = 2
_NS = 16
_SEG_HALF = _NSEG // _NC
_K = 128
_G = _N_EDGES // _K
_IDXROWS = 88
_PH = 80
_GPAD = 2672
_NB = 4
_ZROWS = _SEG_HALF + 8
_RPT = 312


def _sc_body(x_hbm, tn_hbm, ti_hbm, meta_hbm, out_hbm,
             idxn_all, idxs_all, rows0, rows1, rows2, rows3, meta_v, zsh,
             sia, sib, sg0, sg1, sg2, sg3, ss0, ss1, ss2, ss3):
    c = lax.axis_index("c")
    s = lax.axis_index("s")
    rows = (rows0, rows1, rows2, rows3)
    sg = (sg0, sg1, sg2, sg3)
    ss = (ss0, ss1, ss2, ss3)

    pltpu.sync_copy(meta_hbm, meta_v)
    cb_hi = meta_v[pl.ds(0, 16)][0]
    cb_lo = meta_v[pl.ds(16, 16)][0]

    c_lo = jnp.where(c == 0, 0, cb_lo)
    c_cnt = jnp.where(c == 0, cb_hi, _G - cb_lo)
    m = (c_cnt + _NS - 1) // _NS
    S = c_lo + s * m
    n = jnp.clip(c_cnt - s * m, 0, m)
    A = jnp.minimum((S // 8) * 8, _GPAD - _PH - _IDXROWS)
    off = S - A

    cia = pltpu.async_copy(tn_hbm.at[pl.ds(A, _IDXROWS)], idxn_all, sia)
    cib = pltpu.async_copy(ti_hbm.at[pl.ds(A, _IDXROWS)], idxs_all, sib)

    zvec = jnp.zeros((16,), jnp.float32)

    def zrow(i, carry):
        for j in range(_D // 16):
            rows0[i, pl.ds(j * 16, 16)] = zvec
        return carry

    lax.fori_loop(0, _K, zrow, 0)

    zbase = _SEG_HALF * c

    def remap_ids(_):
        def rrow(i, carry):
            for j in range(_D // 16):
                v = idxs_all[i, pl.ds(j * 16, 16)]
                lo = v - zbase
                ok = (lo >= 0) & (lo < _SEG_HALF)
                idxs_all[i, pl.ds(j * 16, 16)] = jnp.where(ok, lo, _SEG_HALF)
            return carry

        lax.fori_loop(0, _IDXROWS, rrow, 0)

    cia.wait()
    cib.wait()
    remap_ids(None)

    r0 = s * _RPT
    _RPT_LAST = _SEG_HALF - (_NS - 1) * _RPT

    def zcopy(nrows):
        pltpu.sync_copy(rows0, zsh.at[pl.ds(r0, _K)])
        pltpu.sync_copy(rows0, zsh.at[pl.ds(r0 + _K, _K)])
        pltpu.sync_copy(rows0.at[pl.ds(0, nrows - 2 * _K)],
                        zsh.at[pl.ds(r0 + 2 * _K, nrows - 2 * _K)])

    @pl.when(s < _NS - 1)
    def _zero_main():
        zcopy(_RPT)

    @pl.when(s == _NS - 1)
    def _zero_last():
        zcopy(_RPT_LAST)
        pltpu.sync_copy(rows0.at[pl.ds(0, 8)],
                        zsh.at[pl.ds(_SEG_HALF, 8)])

    plsc.subcore_barrier()

    def chunk_op(cc, srow_base, b):
        @pl.when(cc >= srow_base + _NB)
        def _drain():
            pltpu.make_async_copy(rows[b], zsh.at[idxs_all.at[off]],
                                  ss[b]).wait()

        row = off + cc - srow_base
        pltpu.async_copy(x_hbm.at[idxn_all.at[row]], rows[b], sg[b]).wait()
        pltpu.async_copy(rows[b], zsh.at[idxs_all.at[row]], ss[b], add=True)

    def step1(t, carry):
        for u in range(_NB):
            cc = _NB * t + u

            @pl.when(cc < n)
            def _chunk(cc=cc, u=u):
                chunk_op(cc, 0, u)

        return carry

    t1_end = jnp.minimum(-(-n // _NB), _PH // _NB)
    lax.fori_loop(0, t1_end, step1, 0)

    @pl.when(n > _PH)
    def _reload():
        for b in range(_NB):
            pltpu.make_async_copy(rows[b], zsh.at[idxs_all.at[off]],
                                  ss[b]).wait()
        pltpu.sync_copy(tn_hbm.at[pl.ds(A + _PH, _IDXROWS)], idxn_all)
        pltpu.sync_copy(ti_hbm.at[pl.ds(A + _PH, _IDXROWS)], idxs_all)
        remap_ids(None)

    def step2(t, carry):
        for u in range(_NB):
            cc = _NB * t + u

            @pl.when(cc < n)
            def _chunk(cc=cc, u=u):
                chunk_op(cc, _PH, u)

        return carry

    t2_end = jnp.where(n > _PH, -(-n // _NB), _PH // _NB)
    lax.fori_loop(_PH // _NB, t2_end, step2, 0)

    nn = jnp.where(n > _PH, n - _PH, n)
    for b in range(_NB):
        @pl.when(b < nn)
        def _drain_tail(b=b):
            pltpu.make_async_copy(rows[b], zsh.at[idxs_all.at[off]],
                                  ss[b]).wait()

    plsc.subcore_barrier()

    @pl.when(s < _NS - 1)
    def _out_main():
        pltpu.sync_copy(zsh.at[pl.ds(r0, _RPT)],
                        out_hbm.at[pl.ds(c * _SEG_HALF + r0, _RPT)])

    @pl.when(s == _NS - 1)
    def _out_last():
        pltpu.sync_copy(zsh.at[pl.ds(r0, _RPT_LAST)],
                        out_hbm.at[pl.ds(c * _SEG_HALF + r0, _RPT_LAST)])


_sc_segment_sum = functools.partial(
    pl.kernel,
    out_type=jax.ShapeDtypeStruct((_NSEG, _D), jnp.float32),
    mesh=plsc.VectorSubcoreMesh(core_axis_name="c", subcore_axis_name="s",
                                num_cores=_NC, num_subcores=_NS),
    scratch_types=[
        pltpu.VMEM((_IDXROWS, _K), jnp.int32),
        pltpu.VMEM((_IDXROWS, _K), jnp.int32),
        pltpu.VMEM((_K, _D), jnp.float32),
        pltpu.VMEM((_K, _D), jnp.float32),
        pltpu.VMEM((_K, _D), jnp.float32),
        pltpu.VMEM((_K, _D), jnp.float32),
        pltpu.VMEM((32,), jnp.int32),
        pltpu.VMEM_SHARED((_ZROWS, _D), jnp.float32),
        pltpu.SemaphoreType.DMA,
        pltpu.SemaphoreType.DMA,
        pltpu.SemaphoreType.DMA,
        pltpu.SemaphoreType.DMA,
        pltpu.SemaphoreType.DMA,
        pltpu.SemaphoreType.DMA,
        pltpu.SemaphoreType.DMA,
        pltpu.SemaphoreType.DMA,
        pltpu.SemaphoreType.DMA,
        pltpu.SemaphoreType.DMA,
    ],
)(_sc_body)


def _mlp_body(zp_ref, w1_ref, b1_ref, w2t_ref, b2_ref, o_ref):
    z = zp_ref[...]
    h = jnp.dot(z, w1_ref[...], preferred_element_type=jnp.float32)
    h = jnp.maximum(h + b1_ref[...], 0.0)
    logit = jnp.sum(h * w2t_ref[...], axis=1, keepdims=True) + b2_ref[...]
    o_ref[...] = jax.nn.sigmoid(logit)


_mlp = pl.pallas_call(
    _mlp_body,
    out_shape=jax.ShapeDtypeStruct((_NSEG, 1), jnp.float32),
)


def kernel(x, target_nodes, target_ids, W1, b1, W2, b2):
    tn1 = target_nodes.astype(jnp.int32)
    ti1 = target_ids.astype(jnp.int32)
    pad = _GPAD * _K - _N_EDGES
    tn = jnp.pad(tn1, (0, pad)).reshape(_GPAD, _K)
    ti = jnp.pad(ti1, (0, pad)).reshape(_GPAD, _K)
    e_star = jnp.searchsorted(ti1, _SEG_HALF).astype(jnp.int32)
    cb_hi = (e_star + _K - 1) // _K
    cb_lo = e_star // _K
    meta = jnp.concatenate([jnp.full((16,), cb_hi, jnp.int32),
                            jnp.full((16,), cb_lo, jnp.int32)])
    z = _sc_segment_sum(x, tn, ti, meta)
    out = _mlp(z, W1, b1.reshape(1, _D), W2.reshape(1, _D), b2.reshape(1, 1))
    return out.reshape(_NSEG)


# speedup vs baseline: 1.1748x; 1.1748x over previous
import functools

import jax
import jax.numpy as jnp
from jax import lax
from jax.experimental import pallas as pl
from jax.experimental.pallas import tpu as pltpu
from jax.experimental.pallas import tpu_sc as plsc

_N_NODES = 10000
_N_EDGES = 320000
_D = 128
_NSEG = 10000
_NC = 2
_NS = 16
_NW = _NC * _NS
_K = 128
_G = _N_EDGES // _K
_NITER = -(-_G // _NW)
_NB = 3
_STEPS = -(-_NITER // _NB)
_RPT = 624


def _sc_body(x_hbm, tn_hbm, ti_hbm, z0_hbm, out_hbm,
             idxn0, idxn1, idxn2, idxs0, idxs1, idxs2,
             rows0, rows1, rows2, zsh,
             sin0, sin1, sin2, sis0, sis1, sis2,
             sg0, sg1, sg2, ss0, ss1, ss2):
    c = lax.axis_index("c")
    s = lax.axis_index("s")
    wid = s * _NC + c
    idxn = (idxn0, idxn1, idxn2)
    idxs = (idxs0, idxs1, idxs2)
    rows = (rows0, rows1, rows2)
    sin = (sin0, sin1, sin2)
    sis = (sis0, sis1, sis2)
    sg = (sg0, sg1, sg2)
    ss = (ss0, ss1, ss2)

    r0 = s * _RPT
    _RPT_LAST = _NSEG - (_NS - 1) * _RPT

    @pl.when(s < _NS - 1)
    def _zero_main():
        pltpu.sync_copy(z0_hbm.at[pl.ds(0, _RPT)], zsh.at[pl.ds(r0, _RPT)])

    @pl.when(s == _NS - 1)
    def _zero_last():
        pltpu.sync_copy(z0_hbm.at[pl.ds(0, _RPT_LAST)],
                        zsh.at[pl.ds(r0, _RPT_LAST)])

    plsc.subcore_barrier()

    def step(t, carry):
        for b in range(_NB):
            cc = _NB * t + b
            g = wid + _NW * cc

            @pl.when(g < _G)
            def _chunk(b=b, cc=cc, g=g):
                e0 = g * _K

                @pl.when(cc >= _NB)
                def _drain_prev():
                    pltpu.make_async_copy(rows[b], zsh.at[idxs[b]], ss[b]).wait()

                cis = pltpu.async_copy(ti_hbm.at[pl.ds(e0, _K)], idxs[b], sis[b])

                @pl.when(cc < _NB)
                def _first_idxn():
                    pltpu.async_copy(tn_hbm.at[pl.ds(e0, _K)], idxn[b], sin[b])

                pltpu.make_async_copy(tn_hbm.at[pl.ds(e0, _K)], idxn[b],
                                      sin[b]).wait()
                pltpu.async_copy(x_hbm.at[idxn[b]], rows[b], sg[b]).wait()
                cis.wait()
                pltpu.async_copy(rows[b], zsh.at[idxs[b]], ss[b], add=True)

                g2 = g + _NB * _NW

                @pl.when(g2 < _G)
                def _prefetch_idxn():
                    pltpu.async_copy(tn_hbm.at[pl.ds(g2 * _K, _K)], idxn[b],
                                     sin[b])

        return carry

    lax.fori_loop(0, _STEPS, step, 0)

    for b in range(_NB):
        pltpu.make_async_copy(rows[b], zsh.at[idxs[b]], ss[b]).wait()
    plsc.subcore_barrier()

    @pl.when(s < _NS - 1)
    def _out_main():
        pltpu.sync_copy(zsh.at[pl.ds(r0, _RPT)],
                        out_hbm.at[pl.ds(c * _NSEG + r0, _RPT)])

    @pl.when(s == _NS - 1)
    def _out_last():
        pltpu.sync_copy(zsh.at[pl.ds(r0, _RPT_LAST)],
                        out_hbm.at[pl.ds(c * _NSEG + r0, _RPT_LAST)])


_sc_segment_sum = functools.partial(
    pl.kernel,
    out_type=jax.ShapeDtypeStruct((_NC * _NSEG, _D), jnp.float32),
    mesh=plsc.VectorSubcoreMesh(core_axis_name="c", subcore_axis_name="s",
                                num_cores=_NC, num_subcores=_NS),
    scratch_types=(
        [pltpu.VMEM((_K,), jnp.int32)] * 6
        + [pltpu.VMEM((_K, _D), jnp.float32)] * 3
        + [pltpu.VMEM_SHARED((_NSEG, _D), jnp.float32)]
        + [pltpu.SemaphoreType.DMA] * 12
    ),
)(_sc_body)


def _mlp_body(zp_ref, w1_ref, b1_ref, w2t_ref, b2_ref, o_ref):
    z = zp_ref[0:_NSEG, :] + zp_ref[_NSEG:2 * _NSEG, :]
    h = jnp.dot(z, w1_ref[...], preferred_element_type=jnp.float32)
    h = jnp.maximum(h + b1_ref[...], 0.0)
    logit = jnp.sum(h * w2t_ref[...], axis=1, keepdims=True) + b2_ref[...]
    o_ref[...] = jax.nn.sigmoid(logit)


_mlp = pl.pallas_call(
    _mlp_body,
    out_shape=jax.ShapeDtypeStruct((_NSEG, 1), jnp.float32),
)


def kernel(x, target_nodes, target_ids, W1, b1, W2, b2):
    tn = target_nodes.astype(jnp.int32)
    ti = target_ids.astype(jnp.int32)
    zeros = jnp.zeros((_NSEG - (_NS - 1) * _RPT, _D), jnp.float32)
    zparts = _sc_segment_sum(x, tn, ti, zeros)
    out = _mlp(zparts, W1, b1.reshape(1, _D), W2.reshape(1, _D),
               b2.reshape(1, 1))
    return out.reshape(_NSEG)


# speedup vs baseline: 1.1793x; 1.0038x over previous
import functools

import jax
import jax.numpy as jnp
from jax import lax
from jax.experimental import pallas as pl
from jax.experimental.pallas import tpu as pltpu
from jax.experimental.pallas import tpu_sc as plsc

_N_NODES = 10000
_N_EDGES = 320000
_D = 128
_NSEG = 10000
_NC = 2
_NS = 16
_NW = _NC * _NS
_K = 128
_G = _N_EDGES // _K
_NITER = -(-_G // _NW)
_HALF = (_NITER + 1) // 2
_RPT = 624


def _sc_body(x_hbm, tn_hbm, ti_hbm, z0_hbm, out_hbm,
             idxn0, idxn1, idxs0, idxs1, rows0, rows1, zsh,
             sin0, sin1, sis0, sis1, sg0, sg1, ss0, ss1):
    c = lax.axis_index("c")
    s = lax.axis_index("s")
    wid = s * _NC + c
    idxn = (idxn0, idxn1)
    idxs = (idxs0, idxs1)
    rows = (rows0, rows1)
    sin = (sin0, sin1)
    sis = (sis0, sis1)
    sg = (sg0, sg1)
    ss = (ss0, ss1)

    r0 = s * _RPT
    _RPT_LAST = _NSEG - (_NS - 1) * _RPT

    @pl.when(s < _NS - 1)
    def _zero_main():
        pltpu.sync_copy(z0_hbm.at[pl.ds(0, _RPT)], zsh.at[pl.ds(r0, _RPT)])

    @pl.when(s == _NS - 1)
    def _zero_last():
        pltpu.sync_copy(z0_hbm.at[pl.ds(0, _RPT_LAST)],
                        zsh.at[pl.ds(r0, _RPT_LAST)])

    plsc.subcore_barrier()

    def step(t, carry):
        for b in range(2):
            cc = 2 * t + b
            g = wid + _NW * cc

            @pl.when(g < _G)
            def _chunk(b=b, cc=cc, g=g):
                e0 = g * _K

                @pl.when(cc >= 2)
                def _drain_prev():
                    pltpu.make_async_copy(rows[b], zsh.at[idxs[b]], ss[b]).wait()

                cis = pltpu.async_copy(ti_hbm.at[pl.ds(e0, _K)], idxs[b], sis[b])

                @pl.when(cc < 2)
                def _first_idxn():
                    pltpu.async_copy(tn_hbm.at[pl.ds(e0, _K)], idxn[b], sin[b])

                pltpu.make_async_copy(tn_hbm.at[pl.ds(e0, _K)], idxn[b],
                                      sin[b]).wait()
                pltpu.async_copy(x_hbm.at[idxn[b]], rows[b], sg[b]).wait()
                cis.wait()
                pltpu.async_copy(rows[b], zsh.at[idxs[b]], ss[b], add=True)

                g2 = g + 2 * _NW

                @pl.when(g2 < _G)
                def _prefetch_idxn():
                    pltpu.async_copy(tn_hbm.at[pl.ds(g2 * _K, _K)], idxn[b],
                                     sin[b])

        return carry

    lax.fori_loop(0, _HALF, step, 0)

    for b in range(2):
        pltpu.make_async_copy(rows[b], zsh.at[idxs[b]], ss[b]).wait()
    plsc.subcore_barrier()

    @pl.when(s < _NS - 1)
    def _out_main():
        pltpu.sync_copy(zsh.at[pl.ds(r0, _RPT)],
                        out_hbm.at[pl.ds(c * _NSEG + r0, _RPT)])

    @pl.when(s == _NS - 1)
    def _out_last():
        pltpu.sync_copy(zsh.at[pl.ds(r0, _RPT_LAST)],
                        out_hbm.at[pl.ds(c * _NSEG + r0, _RPT_LAST)])


_sc_segment_sum = functools.partial(
    pl.kernel,
    out_type=jax.ShapeDtypeStruct((_NC * _NSEG, _D), jnp.float32),
    mesh=plsc.VectorSubcoreMesh(core_axis_name="c", subcore_axis_name="s",
                                num_cores=_NC, num_subcores=_NS),
    scratch_types=[
        pltpu.VMEM((_K,), jnp.int32),
        pltpu.VMEM((_K,), jnp.int32),
        pltpu.VMEM((_K,), jnp.int32),
        pltpu.VMEM((_K,), jnp.int32),
        pltpu.VMEM((_K, _D), jnp.float32),
        pltpu.VMEM((_K, _D), jnp.float32),
        pltpu.VMEM_SHARED((_NSEG, _D), jnp.float32),
        pltpu.SemaphoreType.DMA,
        pltpu.SemaphoreType.DMA,
        pltpu.SemaphoreType.DMA,
        pltpu.SemaphoreType.DMA,
        pltpu.SemaphoreType.DMA,
        pltpu.SemaphoreType.DMA,
        pltpu.SemaphoreType.DMA,
        pltpu.SemaphoreType.DMA,
    ],
)(_sc_body)


def _mlp_body(zp_ref, w1_ref, b1_ref, w2t_ref, b2_ref, o_ref):
    z = zp_ref[0:_NSEG, :] + zp_ref[_NSEG:2 * _NSEG, :]
    h = jnp.dot(z, w1_ref[...], preferred_element_type=jnp.float32)
    h = jnp.maximum(h + b1_ref[...], 0.0)
    logit = jnp.sum(h * w2t_ref[...], axis=1, keepdims=True) + b2_ref[...]
    o_ref[...] = jax.nn.sigmoid(logit)


_mlp = pl.pallas_call(
    _mlp_body,
    out_shape=jax.ShapeDtypeStruct((_NSEG, 1), jnp.float32),
)


def kernel(x, target_nodes, target_ids, W1, b1, W2, b2):
    tn = target_nodes.astype(jnp.int32)
    ti = target_ids.astype(jnp.int32)
    zeros = jnp.zeros((_NSEG - (_NS - 1) * _RPT, _D), jnp.float32)
    zparts = _sc_segment_sum(x, tn, ti, zeros)
    out = _mlp(zparts, W1, b1.reshape(1, _D), W2.reshape(1, _D),
               b2.reshape(1, 1))
    return out.reshape(_NSEG)
